# baseline (speedup 1.0000x reference)
import numpy as np
import jax
import jax.numpy as jnp
from jax.experimental import pallas as pl
from jax.experimental.pallas import tpu as pltpu

_LAMBDA = 0.01
_NBLK = 4

_CH = 16
_RCH = 32
_BCH = 16


def _shrink(v):
    return jnp.where(v > _LAMBDA, v - _LAMBDA,
                     jnp.where(v < -_LAMBDA, v + _LAMBDA, 0.0))


def _dg_lead(m, t):
    return jax.lax.dot_general(m, t, (((1,), (0,)), ((), ())),
                               preferred_element_type=jnp.float32)


def _dg_mid(m, t):
    return jax.lax.dot_general(m, t, (((1,), (1,)), ((), ())),
                               preferred_element_type=jnp.float32)


def _dg_rows(t, m):
    return jax.lax.dot_general(t, m, (((2,), (0,)), ((), ())),
                               preferred_element_type=jnp.float32)


def _body(x_hbm, w1s, w2s, b1, b2, fhs, fws, gws, gh2, tw, out_hbm,
          X, Ar, Ai, Br, Bi, sem_in, sem_out):
    b = pl.program_id(0)
    nb = pl.num_programs(0)

    Hh, Ww, C = X.shape
    bs = C // _NBLK

    FHs = fhs[...]
    FWs = fws[...]
    GWs = gws[...]
    GHre = gh2[0]; GHim = gh2[1]

    @pl.when(b == 0)
    def _():
        pltpu.make_async_copy(x_hbm.at[b], X, sem_in).start()

    pltpu.make_async_copy(x_hbm.at[b], X, sem_in).wait()

    for c0 in range(0, Ww, _CH):
        p = _dg_lead(FHs, X[:, c0:c0 + _CH, :])
        Ar[:, c0:c0 + _CH, :] = p[:Hh]
        Ai[:, c0:c0 + _CH, :] = p[Hh:]

    @pl.when(b + 1 < nb)
    def _():
        pltpu.make_async_copy(x_hbm.at[b + 1], X, sem_in).start()

    for c0 in range(0, Hh, _CH):
        tc = tw[0, c0:c0 + _CH]
        ts = tw[1, c0:c0 + _CH]
        for lg in range(_NBLK):
            lo, hi = lg * bs, (lg + 1) * bs
            ar = Ar[c0:c0 + _CH, :, lo:hi]
            ai = Ai[c0:c0 + _CH, :, lo:hi]
            vr = ar * tc + ai * ts
            vi = ai * tc - ar * ts
            p = _dg_mid(FWs, vr)
            q = _dg_mid(FWs, vi)
            Br[:, c0:c0 + _CH, lo:hi] = p[:Hh] - q[Hh:]
            Bi[:, c0:c0 + _CH, lo:hi] = q[:Hh] + p[Hh:]

    for r0 in range(0, Hh, _BCH):
        r1_ = r0 + _BCH
        z0r = Br[r0:r1_, :, 0 * bs:1 * bs]; z0i = Bi[r0:r1_, :, 0 * bs:1 * bs]
        z1r = Br[r0:r1_, :, 1 * bs:2 * bs]; z1i = Bi[r0:r1_, :, 1 * bs:2 * bs]
        z2r = Br[r0:r1_, :, 2 * bs:3 * bs]; z2i = Bi[r0:r1_, :, 2 * bs:3 * bs]
        z3r = Br[r0:r1_, :, 3 * bs:4 * bs]; z3i = Bi[r0:r1_, :, 3 * bs:4 * bs]
        par = z0r + z2r; pai = z0i + z2i
        pbr = z0r - z2r; pbi = z0i - z2i
        pcr = z1r + z3r; pci = z1i + z3i
        pdr = z1r - z3r; pdi = z1i - z3i
        Ar[r0:r1_, :, 0 * bs:1 * bs] = par + pcr
        Ai[r0:r1_, :, 0 * bs:1 * bs] = pai + pci
        Ar[r0:r1_, :, 1 * bs:2 * bs] = pbr + pdi
        Ai[r0:r1_, :, 1 * bs:2 * bs] = pbi - pdr
        Ar[r0:r1_, :, 2 * bs:3 * bs] = par - pcr
        Ai[r0:r1_, :, 2 * bs:3 * bs] = pai - pci
        Ar[r0:r1_, :, 3 * bs:4 * bs] = pbr - pdi
        Ai[r0:r1_, :, 3 * bs:4 * bs] = pbi + pdr

    for bl in range(_NBLK):
        lo, hi = bl * bs, (bl + 1) * bs
        w1b = w1s[bl]; w2b = w2s[bl]
        b1r = b1[0, bl]; b1i = b1[1, bl]
        b2r = b2[0, bl]; b2i = b2[1, bl]
        for r0 in range(0, Hh, _RCH):
            fr = Ar[r0:r0 + _RCH, :, lo:hi]
            fi = Ai[r0:r0 + _RCH, :, lo:hi]
            p1 = _dg_rows(fr, w1b)
            q1 = _dg_rows(fi, w1b)
            r1 = jnp.maximum(p1[:, :, :bs] - q1[:, :, bs:] + b1r, 0.0)
            i1 = jnp.maximum(p1[:, :, bs:] + q1[:, :, :bs] + b1i, 0.0)
            p2 = _dg_rows(r1, w2b)
            q2 = _dg_rows(i1, w2b)
            r2 = p2[:, :, :bs] - q2[:, :, bs:] + b2r
            i2 = p2[:, :, bs:] + q2[:, :, :bs] + b2i
            Br[r0:r0 + _RCH, :, lo:hi] = _shrink(r2)
            Bi[r0:r0 + _RCH, :, lo:hi] = _shrink(i2)

    for r0 in range(0, Hh, _BCH):
        r1_ = r0 + _BCH
        s0r = Br[r0:r1_, :, 0 * bs:1 * bs]; s0i = Bi[r0:r1_, :, 0 * bs:1 * bs]
        s1r = Br[r0:r1_, :, 1 * bs:2 * bs]; s1i = Bi[r0:r1_, :, 1 * bs:2 * bs]
        s2r = Br[r0:r1_, :, 2 * bs:3 * bs]; s2i = Bi[r0:r1_, :, 2 * bs:3 * bs]
        s3r = Br[r0:r1_, :, 3 * bs:4 * bs]; s3i = Bi[r0:r1_, :, 3 * bs:4 * bs]
        qar = s0r + s2r; qai = s0i + s2i
        qbr = s0r - s2r; qbi = s0i - s2i
        qcr = s1r + s3r; qci = s1i + s3i
        qdr = s1r - s3r; qdi = s1i - s3i
        Ar[r0:r1_, :, 0 * bs:1 * bs] = qar + qcr
        Ai[r0:r1_, :, 0 * bs:1 * bs] = qai + qci
        Ar[r0:r1_, :, 1 * bs:2 * bs] = qbr - qdi
        Ai[r0:r1_, :, 1 * bs:2 * bs] = qbi + qdr
        Ar[r0:r1_, :, 2 * bs:3 * bs] = qar - qcr
        Ai[r0:r1_, :, 2 * bs:3 * bs] = qai - qci
        Ar[r0:r1_, :, 3 * bs:4 * bs] = qbr + qdi
        Ai[r0:r1_, :, 3 * bs:4 * bs] = qbi - qdr

    for c0 in range(0, Hh, _CH):
        zr = Ar[:, c0:c0 + _CH, :]
        zi = Ai[:, c0:c0 + _CH, :]
        p = _dg_lead(GWs, zr)
        q = _dg_lead(GWs, zi)
        Br[:, c0:c0 + _CH, :] = p[:Hh] - q[Hh:]
        Bi[:, c0:c0 + _CH, :] = q[:Hh] + p[Hh:]

    for c0 in range(0, Ww, _CH):
        tc = tw[0, c0:c0 + _CH]
        ts = tw[1, c0:c0 + _CH]
        for lg in range(_NBLK):
            lo, hi = lg * bs, (lg + 1) * bs
            ur = Br[c0:c0 + _CH, :, lo:hi]
            ui = Bi[c0:c0 + _CH, :, lo:hi]
            vr = ur * tc - ui * ts
            vi = ui * tc + ur * ts
            Ar[:, c0:c0 + _CH, lo:hi] = _dg_mid(GHre, vr) - _dg_mid(GHim, vi)

    cout = pltpu.make_async_copy(Ar, out_hbm.at[b], sem_out)
    cout.start()
    cout.wait()


def _tables(Hh, Ww, N, nblk):
    scale = 1.0 / np.sqrt(N * nblk)
    k = np.arange(Hh, dtype=np.float64)
    th1 = 2.0 * np.pi * np.outer(k, k) / Hh
    th2 = 2.0 * np.pi * np.outer(k, k) / N
    c1, s1 = np.cos(th1), np.sin(th1)
    c2, s2 = np.cos(th2), np.sin(th2)
    f32 = np.float32
    fhs = np.concatenate([c1 * scale, -s1 * scale])
    fws = np.concatenate([c1, -s1])
    gws = np.concatenate([c1, s1])
    gh2 = np.stack([c1 * scale, s1 * scale])
    bs = 128
    tw = np.stack([np.repeat(c2[:, :, None], bs, axis=2),
                   np.repeat(s2[:, :, None], bs, axis=2)])
    return (fhs.astype(f32), fws.astype(f32), gws.astype(f32),
            gh2.astype(f32), tw.astype(f32))


def kernel(x, w1, w2, b1, b2, H, W):
    B, N, C = x.shape
    Hh = Ww = 64
    assert N == Hh * Ww and C % _NBLK == 0 and B % 2 == 0
    bs = C // _NBLK

    fhs, fws, gws, gh2, tw = _tables(Hh, Ww, N, _NBLK)
    xr = x.reshape(B, Hh, Ww, C)
    b1r = b1.reshape(2, _NBLK, 1, 1, bs)
    b2r = b2.reshape(2, _NBLK, 1, 1, bs)
    w1s = jnp.concatenate([w1[0], w1[1]], axis=-1)
    w2s = jnp.concatenate([w2[0], w2[1]], axis=-1)

    out = pl.pallas_call(
        _body,
        grid=(B,),
        in_specs=[
            pl.BlockSpec(memory_space=pl.ANY),
            pl.BlockSpec(w1s.shape, lambda i: (0, 0, 0)),
            pl.BlockSpec(w2s.shape, lambda i: (0, 0, 0)),
            pl.BlockSpec(b1r.shape, lambda i: (0, 0, 0, 0, 0)),
            pl.BlockSpec(b2r.shape, lambda i: (0, 0, 0, 0, 0)),
            pl.BlockSpec((2 * Hh, Hh), lambda i: (0, 0)),
            pl.BlockSpec((2 * Hh, Hh), lambda i: (0, 0)),
            pl.BlockSpec((2 * Hh, Hh), lambda i: (0, 0)),
            pl.BlockSpec((2, Hh, Hh), lambda i: (0, 0, 0)),
            pl.BlockSpec((2, Hh, Ww, bs), lambda i: (0, 0, 0, 0)),
        ],
        out_specs=pl.BlockSpec(memory_space=pl.ANY),
        out_shape=jax.ShapeDtypeStruct((B, Hh, Ww, C), jnp.float32),
        scratch_shapes=[
            pltpu.VMEM((Hh, Ww, C), jnp.float32),
            pltpu.VMEM((Hh, Ww, C), jnp.float32),
            pltpu.VMEM((Hh, Ww, C), jnp.float32),
            pltpu.VMEM((Hh, Ww, C), jnp.float32),
            pltpu.VMEM((Hh, Ww, C), jnp.float32),
            pltpu.SemaphoreType.DMA,
            pltpu.SemaphoreType.DMA,
        ],
        compiler_params=pltpu.CompilerParams(
            dimension_semantics=("arbitrary",),
            vmem_limit_bytes=56 * 1024 * 1024,
        ),
        name="einfft_fused",
    )(xr, w1s, w2s, b1r, b2r, jnp.asarray(fhs), jnp.asarray(fws),
      jnp.asarray(gws), jnp.asarray(gh2), jnp.asarray(tw))
    return out.reshape(B, N, C)


# speedup vs baseline: 1.0540x; 1.0540x over previous
import numpy as np
import jax
import jax.numpy as jnp
from jax.experimental import pallas as pl
from jax.experimental.pallas import tpu as pltpu

_LAMBDA = 0.01
_NBLK = 4

_CH = 16
_RCH = 32
_BCH = 16


def _shrink(v):
    return jnp.where(v > _LAMBDA, v - _LAMBDA,
                     jnp.where(v < -_LAMBDA, v + _LAMBDA, 0.0))


_BF = jnp.bfloat16


def _dg_lead(m, t):
    return jax.lax.dot_general(m, t.astype(_BF), (((1,), (0,)), ((), ())),
                               preferred_element_type=jnp.float32)


def _dg_mid(m, t):
    return jax.lax.dot_general(m, t.astype(_BF), (((1,), (1,)), ((), ())),
                               preferred_element_type=jnp.float32)


def _dg_rows(t, m):
    return jax.lax.dot_general(t.astype(_BF), m, (((2,), (0,)), ((), ())),
                               preferred_element_type=jnp.float32)


def _body(x_hbm, w1s, w2s, b1, b2, fhs, fws, gws, gh2, tw, out_hbm,
          X, Ar, Ai, Br, Bi, sem_in, sem_out):
    b = pl.program_id(0)
    nb = pl.num_programs(0)

    Hh, Ww, C = X.shape
    bs = C // _NBLK

    FHs = fhs[...]
    FWs = fws[...]
    GWs = gws[...]
    GHre = gh2[0]; GHim = gh2[1]

    @pl.when(b == 0)
    def _():
        pltpu.make_async_copy(x_hbm.at[b], X, sem_in).start()

    pltpu.make_async_copy(x_hbm.at[b], X, sem_in).wait()

    for c0 in range(0, Ww, _CH):
        p = _dg_lead(FHs, X[:, c0:c0 + _CH, :])
        Ar[:, c0:c0 + _CH, :] = p[:Hh]
        Ai[:, c0:c0 + _CH, :] = p[Hh:]

    @pl.when(b + 1 < nb)
    def _():
        pltpu.make_async_copy(x_hbm.at[b + 1], X, sem_in).start()

    for c0 in range(0, Hh, _CH):
        tc = tw[0, c0:c0 + _CH]
        ts = tw[1, c0:c0 + _CH]
        for lg in range(_NBLK):
            lo, hi = lg * bs, (lg + 1) * bs
            ar = Ar[c0:c0 + _CH, :, lo:hi]
            ai = Ai[c0:c0 + _CH, :, lo:hi]
            vr = ar * tc + ai * ts
            vi = ai * tc - ar * ts
            p = _dg_mid(FWs, vr)
            q = _dg_mid(FWs, vi)
            Br[:, c0:c0 + _CH, lo:hi] = p[:Hh] - q[Hh:]
            Bi[:, c0:c0 + _CH, lo:hi] = q[:Hh] + p[Hh:]

    for r0 in range(0, Hh, _BCH):
        r1_ = r0 + _BCH
        z0r = Br[r0:r1_, :, 0 * bs:1 * bs]; z0i = Bi[r0:r1_, :, 0 * bs:1 * bs]
        z1r = Br[r0:r1_, :, 1 * bs:2 * bs]; z1i = Bi[r0:r1_, :, 1 * bs:2 * bs]
        z2r = Br[r0:r1_, :, 2 * bs:3 * bs]; z2i = Bi[r0:r1_, :, 2 * bs:3 * bs]
        z3r = Br[r0:r1_, :, 3 * bs:4 * bs]; z3i = Bi[r0:r1_, :, 3 * bs:4 * bs]
        par = z0r + z2r; pai = z0i + z2i
        pbr = z0r - z2r; pbi = z0i - z2i
        pcr = z1r + z3r; pci = z1i + z3i
        pdr = z1r - z3r; pdi = z1i - z3i
        Ar[r0:r1_, :, 0 * bs:1 * bs] = par + pcr
        Ai[r0:r1_, :, 0 * bs:1 * bs] = pai + pci
        Ar[r0:r1_, :, 1 * bs:2 * bs] = pbr + pdi
        Ai[r0:r1_, :, 1 * bs:2 * bs] = pbi - pdr
        Ar[r0:r1_, :, 2 * bs:3 * bs] = par - pcr
        Ai[r0:r1_, :, 2 * bs:3 * bs] = pai - pci
        Ar[r0:r1_, :, 3 * bs:4 * bs] = pbr - pdi
        Ai[r0:r1_, :, 3 * bs:4 * bs] = pbi + pdr

    for bl in range(_NBLK):
        lo, hi = bl * bs, (bl + 1) * bs
        w1b = w1s[bl]; w2b = w2s[bl]
        b1r = b1[0, bl]; b1i = b1[1, bl]
        b2r = b2[0, bl]; b2i = b2[1, bl]
        for r0 in range(0, Hh, _RCH):
            fr = Ar[r0:r0 + _RCH, :, lo:hi]
            fi = Ai[r0:r0 + _RCH, :, lo:hi]
            p1 = _dg_rows(fr, w1b)
            q1 = _dg_rows(fi, w1b)
            r1 = jnp.maximum(p1[:, :, :bs] - q1[:, :, bs:] + b1r, 0.0)
            i1 = jnp.maximum(p1[:, :, bs:] + q1[:, :, :bs] + b1i, 0.0)
            p2 = _dg_rows(r1, w2b)
            q2 = _dg_rows(i1, w2b)
            r2 = p2[:, :, :bs] - q2[:, :, bs:] + b2r
            i2 = p2[:, :, bs:] + q2[:, :, :bs] + b2i
            Br[r0:r0 + _RCH, :, lo:hi] = _shrink(r2)
            Bi[r0:r0 + _RCH, :, lo:hi] = _shrink(i2)

    for r0 in range(0, Hh, _BCH):
        r1_ = r0 + _BCH
        s0r = Br[r0:r1_, :, 0 * bs:1 * bs]; s0i = Bi[r0:r1_, :, 0 * bs:1 * bs]
        s1r = Br[r0:r1_, :, 1 * bs:2 * bs]; s1i = Bi[r0:r1_, :, 1 * bs:2 * bs]
        s2r = Br[r0:r1_, :, 2 * bs:3 * bs]; s2i = Bi[r0:r1_, :, 2 * bs:3 * bs]
        s3r = Br[r0:r1_, :, 3 * bs:4 * bs]; s3i = Bi[r0:r1_, :, 3 * bs:4 * bs]
        qar = s0r + s2r; qai = s0i + s2i
        qbr = s0r - s2r; qbi = s0i - s2i
        qcr = s1r + s3r; qci = s1i + s3i
        qdr = s1r - s3r; qdi = s1i - s3i
        Ar[r0:r1_, :, 0 * bs:1 * bs] = qar + qcr
        Ai[r0:r1_, :, 0 * bs:1 * bs] = qai + qci
        Ar[r0:r1_, :, 1 * bs:2 * bs] = qbr - qdi
        Ai[r0:r1_, :, 1 * bs:2 * bs] = qbi + qdr
        Ar[r0:r1_, :, 2 * bs:3 * bs] = qar - qcr
        Ai[r0:r1_, :, 2 * bs:3 * bs] = qai - qci
        Ar[r0:r1_, :, 3 * bs:4 * bs] = qbr + qdi
        Ai[r0:r1_, :, 3 * bs:4 * bs] = qbi - qdr

    for c0 in range(0, Hh, _CH):
        zr = Ar[:, c0:c0 + _CH, :]
        zi = Ai[:, c0:c0 + _CH, :]
        p = _dg_lead(GWs, zr)
        q = _dg_lead(GWs, zi)
        Br[:, c0:c0 + _CH, :] = p[:Hh] - q[Hh:]
        Bi[:, c0:c0 + _CH, :] = q[:Hh] + p[Hh:]

    for c0 in range(0, Ww, _CH):
        tc = tw[0, c0:c0 + _CH]
        ts = tw[1, c0:c0 + _CH]
        for lg in range(_NBLK):
            lo, hi = lg * bs, (lg + 1) * bs
            ur = Br[c0:c0 + _CH, :, lo:hi]
            ui = Bi[c0:c0 + _CH, :, lo:hi]
            vr = ur * tc - ui * ts
            vi = ui * tc + ur * ts
            Ar[:, c0:c0 + _CH, lo:hi] = _dg_mid(GHre, vr) - _dg_mid(GHim, vi)

    cout = pltpu.make_async_copy(Ar, out_hbm.at[b], sem_out)
    cout.start()
    cout.wait()


def _tables(Hh, Ww, N, nblk):
    scale = 1.0 / np.sqrt(N * nblk)
    k = np.arange(Hh, dtype=np.float64)
    th1 = 2.0 * np.pi * np.outer(k, k) / Hh
    th2 = 2.0 * np.pi * np.outer(k, k) / N
    c1, s1 = np.cos(th1), np.sin(th1)
    c2, s2 = np.cos(th2), np.sin(th2)
    f32 = np.float32
    fhs = np.concatenate([c1, -s1])
    fws = np.concatenate([c1, -s1])
    gws = np.concatenate([c1, s1])
    gh2 = np.stack([c1, s1])
    bs = 128
    rep = lambda a: np.repeat(a[:, :, None], bs, axis=2)
    tw = np.stack([rep(c2 * scale), rep(s2 * scale)])
    return (fhs.astype(f32), fws.astype(f32), gws.astype(f32),
            gh2.astype(f32), tw.astype(f32))


def kernel(x, w1, w2, b1, b2, H, W):
    B, N, C = x.shape
    Hh = Ww = 64
    assert N == Hh * Ww and C % _NBLK == 0 and B % 2 == 0
    bs = C // _NBLK

    fhs, fws, gws, gh2, tw = _tables(Hh, Ww, N, _NBLK)
    xr = x.reshape(B, Hh, Ww, C)
    b1r = b1.reshape(2, _NBLK, 1, 1, bs)
    b2r = b2.reshape(2, _NBLK, 1, 1, bs)
    w1s = jnp.concatenate([w1[0], w1[1]], axis=-1).astype(_BF)
    w2s = jnp.concatenate([w2[0], w2[1]], axis=-1).astype(_BF)

    out = pl.pallas_call(
        _body,
        grid=(B,),
        in_specs=[
            pl.BlockSpec(memory_space=pl.ANY),
            pl.BlockSpec(w1s.shape, lambda i: (0, 0, 0)),
            pl.BlockSpec(w2s.shape, lambda i: (0, 0, 0)),
            pl.BlockSpec(b1r.shape, lambda i: (0, 0, 0, 0, 0)),
            pl.BlockSpec(b2r.shape, lambda i: (0, 0, 0, 0, 0)),
            pl.BlockSpec((2 * Hh, Hh), lambda i: (0, 0)),
            pl.BlockSpec((2 * Hh, Hh), lambda i: (0, 0)),
            pl.BlockSpec((2 * Hh, Hh), lambda i: (0, 0)),
            pl.BlockSpec((2, Hh, Hh), lambda i: (0, 0, 0)),
            pl.BlockSpec((2, Hh, Ww, bs), lambda i: (0, 0, 0, 0)),
        ],
        out_specs=pl.BlockSpec(memory_space=pl.ANY),
        out_shape=jax.ShapeDtypeStruct((B, Hh, Ww, C), jnp.float32),
        scratch_shapes=[
            pltpu.VMEM((Hh, Ww, C), jnp.float32),
            pltpu.VMEM((Hh, Ww, C), jnp.float32),
            pltpu.VMEM((Hh, Ww, C), jnp.float32),
            pltpu.VMEM((Hh, Ww, C), jnp.float32),
            pltpu.VMEM((Hh, Ww, C), jnp.float32),
            pltpu.SemaphoreType.DMA,
            pltpu.SemaphoreType.DMA,
        ],
        compiler_params=pltpu.CompilerParams(
            dimension_semantics=("arbitrary",),
            vmem_limit_bytes=56 * 1024 * 1024,
        ),
        name="einfft_fused",
    )(xr, w1s, w2s, b1r, b2r,
      jnp.asarray(fhs).astype(_BF), jnp.asarray(fws).astype(_BF),
      jnp.asarray(gws).astype(_BF), jnp.asarray(gh2).astype(_BF),
      jnp.asarray(tw))
    return out.reshape(B, N, C)


# speedup vs baseline: 1.0596x; 1.0053x over previous
import numpy as np
import jax
import jax.numpy as jnp
from jax.experimental import pallas as pl
from jax.experimental.pallas import tpu as pltpu

_LAMBDA = 0.01
_NBLK = 4

_CH = 16
_RCH = 16


def _shrink(v):
    return jnp.where(v > _LAMBDA, v - _LAMBDA,
                     jnp.where(v < -_LAMBDA, v + _LAMBDA, 0.0))


_BF = jnp.bfloat16


def _dg_lead(m, t):
    return jax.lax.dot_general(m, t.astype(_BF), (((1,), (0,)), ((), ())),
                               preferred_element_type=jnp.float32)


def _dg_mid(m, t):
    return jax.lax.dot_general(m, t.astype(_BF), (((1,), (1,)), ((), ())),
                               preferred_element_type=jnp.float32)


def _dg_rows(t, m):
    return jax.lax.dot_general(t.astype(_BF), m, (((2,), (0,)), ((), ())),
                               preferred_element_type=jnp.float32)


def _body(x_hbm, w1s, w2s, b1, b2, fhs, fws, gws, gh2, tw, out_hbm,
          X, Ar, Ai, Br, Bi, sem_in, sem_out):
    b = pl.program_id(0)
    nb = pl.num_programs(0)

    Hh, Ww, C = X.shape
    bs = C // _NBLK

    FHs = fhs[...]
    FWs = fws[...]
    GWs = gws[...]
    GHre = gh2[0]; GHim = gh2[1]

    @pl.when(b == 0)
    def _():
        pltpu.make_async_copy(x_hbm.at[b], X, sem_in).start()

    pltpu.make_async_copy(x_hbm.at[b], X, sem_in).wait()

    for c0 in range(0, Ww, _CH):
        p = _dg_lead(FHs, X[:, c0:c0 + _CH, :])
        Ar[:, c0:c0 + _CH, :] = p[:Hh]
        Ai[:, c0:c0 + _CH, :] = p[Hh:]

    @pl.when(b + 1 < nb)
    def _():
        pltpu.make_async_copy(x_hbm.at[b + 1], X, sem_in).start()

    for c0 in range(0, Hh, _CH):
        tc = tw[0, c0:c0 + _CH]
        ts = tw[1, c0:c0 + _CH]
        for lg in range(_NBLK):
            lo, hi = lg * bs, (lg + 1) * bs
            ar = Ar[c0:c0 + _CH, :, lo:hi]
            ai = Ai[c0:c0 + _CH, :, lo:hi]
            vr = ar * tc + ai * ts
            vi = ai * tc - ar * ts
            p = _dg_mid(FWs, vr)
            q = _dg_mid(FWs, vi)
            Br[:, c0:c0 + _CH, lo:hi] = p[:Hh] - q[Hh:]
            Bi[:, c0:c0 + _CH, lo:hi] = q[:Hh] + p[Hh:]

    for r0 in range(0, Hh, _RCH):
        r1_ = r0 + _RCH
        z0r = Br[r0:r1_, :, 0 * bs:1 * bs]; z0i = Bi[r0:r1_, :, 0 * bs:1 * bs]
        z1r = Br[r0:r1_, :, 1 * bs:2 * bs]; z1i = Bi[r0:r1_, :, 1 * bs:2 * bs]
        z2r = Br[r0:r1_, :, 2 * bs:3 * bs]; z2i = Bi[r0:r1_, :, 2 * bs:3 * bs]
        z3r = Br[r0:r1_, :, 3 * bs:4 * bs]; z3i = Bi[r0:r1_, :, 3 * bs:4 * bs]
        par = z0r + z2r; pai = z0i + z2i
        pbr = z0r - z2r; pbi = z0i - z2i
        pcr = z1r + z3r; pci = z1i + z3i
        pdr = z1r - z3r; pdi = z1i - z3i
        fblocks = ((par + pcr, pai + pci), (pbr + pdi, pbi - pdr),
                   (par - pcr, pai - pci), (pbr - pdi, pbi + pdr))
        for bl in range(_NBLK):
            lo, hi = bl * bs, (bl + 1) * bs
            fr, fi = fblocks[bl]
            w1b = w1s[bl]; w2b = w2s[bl]
            b1r = b1[0, bl]; b1i = b1[1, bl]
            b2r = b2[0, bl]; b2i = b2[1, bl]
            p1 = _dg_rows(fr, w1b)
            q1 = _dg_rows(fi, w1b)
            r1 = jnp.maximum(p1[:, :, :bs] - q1[:, :, bs:] + b1r, 0.0)
            i1 = jnp.maximum(p1[:, :, bs:] + q1[:, :, :bs] + b1i, 0.0)
            p2 = _dg_rows(r1, w2b)
            q2 = _dg_rows(i1, w2b)
            r2 = p2[:, :, :bs] - q2[:, :, bs:] + b2r
            i2 = p2[:, :, bs:] + q2[:, :, :bs] + b2i
            Ar[r0:r1_, :, lo:hi] = _shrink(r2)
            Ai[r0:r1_, :, lo:hi] = _shrink(i2)

    for c0 in range(0, Hh, _CH):
        c1_ = c0 + _CH
        s0r = Ar[:, c0:c1_, 0 * bs:1 * bs]; s0i = Ai[:, c0:c1_, 0 * bs:1 * bs]
        s1r = Ar[:, c0:c1_, 1 * bs:2 * bs]; s1i = Ai[:, c0:c1_, 1 * bs:2 * bs]
        s2r = Ar[:, c0:c1_, 2 * bs:3 * bs]; s2i = Ai[:, c0:c1_, 2 * bs:3 * bs]
        s3r = Ar[:, c0:c1_, 3 * bs:4 * bs]; s3i = Ai[:, c0:c1_, 3 * bs:4 * bs]
        qar = s0r + s2r; qai = s0i + s2i
        qbr = s0r - s2r; qbi = s0i - s2i
        qcr = s1r + s3r; qci = s1i + s3i
        qdr = s1r - s3r; qdi = s1i - s3i
        zblocks = ((qar + qcr, qai + qci), (qbr - qdi, qbi + qdr),
                   (qar - qcr, qai - qci), (qbr + qdi, qbi - qdr))
        for lg in range(_NBLK):
            lo, hi = lg * bs, (lg + 1) * bs
            zr, zi = zblocks[lg]
            p = _dg_lead(GWs, zr)
            q = _dg_lead(GWs, zi)
            Br[:, c0:c1_, lo:hi] = p[:Hh] - q[Hh:]
            Bi[:, c0:c1_, lo:hi] = q[:Hh] + p[Hh:]

    for c0 in range(0, Ww, _CH):
        tc = tw[0, c0:c0 + _CH]
        ts = tw[1, c0:c0 + _CH]
        for lg in range(_NBLK):
            lo, hi = lg * bs, (lg + 1) * bs
            ur = Br[c0:c0 + _CH, :, lo:hi]
            ui = Bi[c0:c0 + _CH, :, lo:hi]
            vr = ur * tc - ui * ts
            vi = ui * tc + ur * ts
            Ar[:, c0:c0 + _CH, lo:hi] = _dg_mid(GHre, vr) - _dg_mid(GHim, vi)

    cout = pltpu.make_async_copy(Ar, out_hbm.at[b], sem_out)
    cout.start()
    cout.wait()


def _tables(Hh, Ww, N, nblk):
    scale = 1.0 / np.sqrt(N * nblk)
    k = np.arange(Hh, dtype=np.float64)
    th1 = 2.0 * np.pi * np.outer(k, k) / Hh
    th2 = 2.0 * np.pi * np.outer(k, k) / N
    c1, s1 = np.cos(th1), np.sin(th1)
    c2, s2 = np.cos(th2), np.sin(th2)
    f32 = np.float32
    fhs = np.concatenate([c1, -s1])
    fws = np.concatenate([c1, -s1])
    gws = np.concatenate([c1, s1])
    gh2 = np.stack([c1, s1])
    bs = 128
    rep = lambda a: np.repeat(a[:, :, None], bs, axis=2)
    tw = np.stack([rep(c2 * scale), rep(s2 * scale)])
    return (fhs.astype(f32), fws.astype(f32), gws.astype(f32),
            gh2.astype(f32), tw.astype(f32))


def kernel(x, w1, w2, b1, b2, H, W):
    B, N, C = x.shape
    Hh = Ww = 64
    assert N == Hh * Ww and C % _NBLK == 0 and B % 2 == 0
    bs = C // _NBLK

    fhs, fws, gws, gh2, tw = _tables(Hh, Ww, N, _NBLK)
    xr = x.reshape(B, Hh, Ww, C)
    b1r = b1.reshape(2, _NBLK, 1, 1, bs)
    b2r = b2.reshape(2, _NBLK, 1, 1, bs)
    w1s = jnp.concatenate([w1[0], w1[1]], axis=-1).astype(_BF)
    w2s = jnp.concatenate([w2[0], w2[1]], axis=-1).astype(_BF)

    out = pl.pallas_call(
        _body,
        grid=(B,),
        in_specs=[
            pl.BlockSpec(memory_space=pl.ANY),
            pl.BlockSpec(w1s.shape, lambda i: (0, 0, 0)),
            pl.BlockSpec(w2s.shape, lambda i: (0, 0, 0)),
            pl.BlockSpec(b1r.shape, lambda i: (0, 0, 0, 0, 0)),
            pl.BlockSpec(b2r.shape, lambda i: (0, 0, 0, 0, 0)),
            pl.BlockSpec((2 * Hh, Hh), lambda i: (0, 0)),
            pl.BlockSpec((2 * Hh, Hh), lambda i: (0, 0)),
            pl.BlockSpec((2 * Hh, Hh), lambda i: (0, 0)),
            pl.BlockSpec((2, Hh, Hh), lambda i: (0, 0, 0)),
            pl.BlockSpec((2, Hh, Ww, bs), lambda i: (0, 0, 0, 0)),
        ],
        out_specs=pl.BlockSpec(memory_space=pl.ANY),
        out_shape=jax.ShapeDtypeStruct((B, Hh, Ww, C), jnp.float32),
        scratch_shapes=[
            pltpu.VMEM((Hh, Ww, C), jnp.float32),
            pltpu.VMEM((Hh, Ww, C), jnp.float32),
            pltpu.VMEM((Hh, Ww, C), jnp.float32),
            pltpu.VMEM((Hh, Ww, C), jnp.float32),
            pltpu.VMEM((Hh, Ww, C), jnp.float32),
            pltpu.SemaphoreType.DMA,
            pltpu.SemaphoreType.DMA,
        ],
        compiler_params=pltpu.CompilerParams(
            dimension_semantics=("arbitrary",),
            vmem_limit_bytes=56 * 1024 * 1024,
        ),
        name="einfft_fused",
    )(xr, w1s, w2s, b1r, b2r,
      jnp.asarray(fhs).astype(_BF), jnp.asarray(fws).astype(_BF),
      jnp.asarray(gws).astype(_BF), jnp.asarray(gh2).astype(_BF),
      jnp.asarray(tw))
    return out.reshape(B, N, C)
